# baseline (speedup 1.0000x reference)
import functools

import jax
import jax.numpy as jnp
from jax.experimental import pallas as pl
from jax.experimental.pallas import tpu as pltpu

NUM_EXPERTS = 64
TOP_K = 8
HIDDEN = 4096
BLOCK_T = 1024


def _router_kernel(x0_ref, x1_ref, x2_ref, x3_ref, wt_hbm_ref, b_ref,
                   rw_ref, mask_ref, wt_ref, sem):
    @pl.when(pl.program_id(0) == 0)
    def _():
        copy = pltpu.make_async_copy(wt_hbm_ref, wt_ref, sem)
        copy.start()
        copy.wait()

    wt = wt_ref[...]
    logits = jnp.concatenate(
        [jnp.dot(r[...], wt, preferred_element_type=jnp.float32,
                 precision=jax.lax.Precision.DEFAULT)
         for r in (x0_ref, x1_ref, x2_ref, x3_ref)], axis=0)
    logits = logits + b_ref[...]

    lt = logits.T
    iota = jax.lax.broadcasted_iota(jnp.int32, lt.shape, 0)
    work = lt
    selected = jnp.zeros(lt.shape, dtype=jnp.bool_)
    for _ in range(TOP_K):
        m = jnp.max(work, axis=0, keepdims=True)
        is_max = work == m
        first = jnp.min(jnp.where(is_max, iota, NUM_EXPERTS),
                        axis=0, keepdims=True)
        sel = iota == first
        selected = selected | sel
        work = jnp.where(sel, -jnp.inf, work)

    gmax = jnp.max(lt, axis=0, keepdims=True)
    e = jnp.where(selected, jnp.exp(lt - gmax), 0.0)
    rw = e / jnp.sum(e, axis=0, keepdims=True)
    rw_ref[...] = rw.T
    mask_ref[...] = selected.astype(jnp.float32).T


@functools.partial(jax.jit, static_argnames=())
def kernel(hidden_states, W, b):
    B, S, H = hidden_states.shape
    T = B * S
    x = hidden_states.reshape(T, H)
    wt = W.T
    b2 = b.reshape(1, NUM_EXPERTS)

    grid = (T // BLOCK_T,)
    QT = BLOCK_T // 4
    rw, mask = pl.pallas_call(
        _router_kernel,
        grid=grid,
        in_specs=[
            pl.BlockSpec((QT, H), lambda i: (4 * i, 0)),
            pl.BlockSpec((QT, H), lambda i: (4 * i + 1, 0)),
            pl.BlockSpec((QT, H), lambda i: (4 * i + 2, 0)),
            pl.BlockSpec((QT, H), lambda i: (4 * i + 3, 0)),
            pl.BlockSpec(memory_space=pltpu.MemorySpace.HBM),
            pl.BlockSpec((1, NUM_EXPERTS), lambda i: (0, 0)),
        ],
        out_specs=[
            pl.BlockSpec((BLOCK_T, NUM_EXPERTS), lambda i: (i, 0)),
            pl.BlockSpec((BLOCK_T, NUM_EXPERTS), lambda i: (i, 0)),
        ],
        out_shape=[
            jax.ShapeDtypeStruct((T, NUM_EXPERTS), jnp.float32),
            jax.ShapeDtypeStruct((T, NUM_EXPERTS), jnp.float32),
        ],
        scratch_shapes=[
            pltpu.VMEM((HIDDEN, NUM_EXPERTS), jnp.float32),
            pltpu.SemaphoreType.DMA,
        ],
    )(x, x, x, x, wt, b2)
    return (rw.reshape(B, S, NUM_EXPERTS), mask.reshape(B, S, NUM_EXPERTS))


# speedup vs baseline: 1.0598x; 1.0598x over previous
import functools

import jax
import jax.numpy as jnp
from jax.experimental import pallas as pl

NUM_EXPERTS = 64
TOP_K = 8
HIDDEN = 4096
BLOCK_T = 1024


def _router_kernel(x0_ref, x1_ref, x2_ref, x3_ref, wt_ref, b_ref,
                   rw_ref, mask_ref):
    wt = wt_ref[...]
    logits = jnp.concatenate(
        [jnp.dot(r[...], wt, preferred_element_type=jnp.float32,
                 precision=jax.lax.Precision.DEFAULT)
         for r in (x0_ref, x1_ref, x2_ref, x3_ref)], axis=0)
    logits = logits + b_ref[...]

    lt = logits.T
    iota = jax.lax.broadcasted_iota(jnp.int32, lt.shape, 0)
    work = lt
    selected = jnp.zeros(lt.shape, dtype=jnp.bool_)
    for _ in range(TOP_K):
        m = jnp.max(work, axis=0, keepdims=True)
        is_max = work == m
        first = jnp.min(jnp.where(is_max, iota, NUM_EXPERTS),
                        axis=0, keepdims=True)
        sel = iota == first
        selected = selected | sel
        work = jnp.where(sel, -jnp.inf, work)

    gmax = jnp.max(lt, axis=0, keepdims=True)
    e = jnp.where(selected, jnp.exp(lt - gmax), 0.0)
    rw = e / jnp.sum(e, axis=0, keepdims=True)
    rw_ref[...] = rw.T
    mask_ref[...] = selected.astype(jnp.float32).T


@functools.partial(jax.jit, static_argnames=())
def kernel(hidden_states, W, b):
    B, S, H = hidden_states.shape
    T = B * S
    x = hidden_states.reshape(T, H)
    wt = W.T
    b2 = b.reshape(1, NUM_EXPERTS)

    grid = (T // BLOCK_T,)
    QT = BLOCK_T // 4
    rw, mask = pl.pallas_call(
        _router_kernel,
        grid=grid,
        in_specs=[
            pl.BlockSpec((QT, H), lambda i: (4 * i, 0)),
            pl.BlockSpec((QT, H), lambda i: (4 * i + 1, 0)),
            pl.BlockSpec((QT, H), lambda i: (4 * i + 2, 0)),
            pl.BlockSpec((QT, H), lambda i: (4 * i + 3, 0)),
            pl.BlockSpec((H, NUM_EXPERTS), lambda i: (0, 0)),
            pl.BlockSpec((1, NUM_EXPERTS), lambda i: (0, 0)),
        ],
        out_specs=[
            pl.BlockSpec((BLOCK_T, NUM_EXPERTS), lambda i: (i, 0)),
            pl.BlockSpec((BLOCK_T, NUM_EXPERTS), lambda i: (i, 0)),
        ],
        out_shape=[
            jax.ShapeDtypeStruct((T, NUM_EXPERTS), jnp.float32),
            jax.ShapeDtypeStruct((T, NUM_EXPERTS), jnp.float32),
        ],
    )(x, x, x, x, wt, b2)
    return (rw.reshape(B, S, NUM_EXPERTS), mask.reshape(B, S, NUM_EXPERTS))


# speedup vs baseline: 1.0778x; 1.0170x over previous
import functools

import jax
import jax.numpy as jnp
from jax.experimental import pallas as pl

NUM_EXPERTS = 64
TOP_K = 8
HIDDEN = 4096
BLOCK_T = 1024


def _router_kernel(x0_ref, x1_ref, x2_ref, x3_ref, wt_ref, b_ref,
                   rw_ref, mask_ref):
    wt = wt_ref[...]
    logits = jnp.concatenate(
        [jnp.dot(r[...].astype(jnp.bfloat16), wt,
                 preferred_element_type=jnp.float32)
         for r in (x0_ref, x1_ref, x2_ref, x3_ref)], axis=0)
    logits = logits + b_ref[...]

    lt = logits.T
    iota = jax.lax.broadcasted_iota(jnp.int32, lt.shape, 0)
    work = lt
    selected = jnp.zeros(lt.shape, dtype=jnp.bool_)
    for _ in range(TOP_K):
        m = jnp.max(work, axis=0, keepdims=True)
        is_max = work == m
        first = jnp.min(jnp.where(is_max, iota, NUM_EXPERTS),
                        axis=0, keepdims=True)
        sel = iota == first
        selected = selected | sel
        work = jnp.where(sel, -jnp.inf, work)

    gmax = jnp.max(lt, axis=0, keepdims=True)
    e = jnp.where(selected, jnp.exp(lt - gmax), 0.0)
    rw = e / jnp.sum(e, axis=0, keepdims=True)
    rw_ref[...] = rw.T
    mask_ref[...] = selected.astype(jnp.float32).T


@functools.partial(jax.jit, static_argnames=())
def kernel(hidden_states, W, b):
    B, S, H = hidden_states.shape
    T = B * S
    x = hidden_states.reshape(T, H)
    wt = W.T.astype(jnp.bfloat16)
    b2 = b.reshape(1, NUM_EXPERTS)

    grid = (T // BLOCK_T,)
    QT = BLOCK_T // 4
    rw, mask = pl.pallas_call(
        _router_kernel,
        grid=grid,
        in_specs=[
            pl.BlockSpec((QT, H), lambda i: (4 * i, 0)),
            pl.BlockSpec((QT, H), lambda i: (4 * i + 1, 0)),
            pl.BlockSpec((QT, H), lambda i: (4 * i + 2, 0)),
            pl.BlockSpec((QT, H), lambda i: (4 * i + 3, 0)),
            pl.BlockSpec((H, NUM_EXPERTS), lambda i: (0, 0)),
            pl.BlockSpec((1, NUM_EXPERTS), lambda i: (0, 0)),
        ],
        out_specs=[
            pl.BlockSpec((BLOCK_T, NUM_EXPERTS), lambda i: (i, 0)),
            pl.BlockSpec((BLOCK_T, NUM_EXPERTS), lambda i: (i, 0)),
        ],
        out_shape=[
            jax.ShapeDtypeStruct((T, NUM_EXPERTS), jnp.float32),
            jax.ShapeDtypeStruct((T, NUM_EXPERTS), jnp.float32),
        ],
    )(x, x, x, x, wt, b2)
    return (rw.reshape(B, S, NUM_EXPERTS), mask.reshape(B, S, NUM_EXPERTS))
